# baseline (speedup 1.0000x reference)
import functools

import jax
import jax.numpy as jnp
from jax import lax
from jax.experimental import pallas as pl
from jax.experimental.pallas import tpu as pltpu, tpu_sc as plsc

NUM_CORES = 2
NUM_SUBCORES = 16
NUM_WORKERS = NUM_CORES * NUM_SUBCORES
LANES = 16

B = 16384
D = 64
B_PER_W = B // NUM_WORKERS
IDX_CHUNK = 128
N_CHUNKS = B_PER_W // IDX_CHUNK
D_VECS = D // LANES


def _distmult_body(si_hbm, pi_hbm, oi_hbm, nodes_hbm, rel_hbm, out_hbm,
                   si_v, pi_v, oi_v, s_rows, p_rows, o_rows, out_v, sem):
    wid = lax.axis_index("s") * NUM_CORES + lax.axis_index("c")
    base = wid * B_PER_W

    for c in range(N_CHUNKS):
        off = base + c * IDX_CHUNK
        pltpu.sync_copy(si_hbm.at[pl.ds(off, IDX_CHUNK)], si_v.at[c])
        pltpu.sync_copy(pi_hbm.at[pl.ds(off, IDX_CHUNK)], pi_v.at[c])
        pltpu.sync_copy(oi_hbm.at[pl.ds(off, IDX_CHUNK)], oi_v.at[c])

    copies = []
    for c in range(N_CHUNKS):
        rows = pl.ds(c * IDX_CHUNK, IDX_CHUNK)
        copies.append(pltpu.make_async_copy(nodes_hbm.at[si_v.at[c]],
                                            s_rows.at[rows], sem))
        copies.append(pltpu.make_async_copy(rel_hbm.at[pi_v.at[c]],
                                            p_rows.at[rows], sem))
        copies.append(pltpu.make_async_copy(nodes_hbm.at[oi_v.at[c]],
                                            o_rows.at[rows], sem))
    for cp in copies:
        cp.start()
    for cp in copies:
        cp.wait()

    lane = lax.iota(jnp.int32, LANES)

    def group(g, _):
        out16 = jnp.zeros((LANES,), jnp.float32)
        for r in range(LANES):
            b = g * LANES + r
            acc = (s_rows[b, pl.ds(0, LANES)]
                   * p_rows[b, pl.ds(0, LANES)]
                   * o_rows[b, pl.ds(0, LANES)])
            for j in range(1, D_VECS):
                sl = pl.ds(j * LANES, LANES)
                acc = acc + s_rows[b, sl] * p_rows[b, sl] * o_rows[b, sl]
            out16 = jnp.where(lane == r, jnp.sum(acc), out16)
        out_v[pl.ds(g * LANES, LANES)] = out16
        return 0

    lax.fori_loop(0, B_PER_W // LANES, group, 0)

    pltpu.sync_copy(out_v, out_hbm.at[pl.ds(base, B_PER_W)])


@jax.jit
def _distmult(si, pi, oi, nodes, relations):
    mesh = plsc.VectorSubcoreMesh(core_axis_name="c", subcore_axis_name="s")
    return pl.kernel(
        _distmult_body,
        out_type=jax.ShapeDtypeStruct((B,), jnp.float32),
        mesh=mesh,
        scratch_types=[
            pltpu.VMEM((N_CHUNKS, IDX_CHUNK), jnp.int32),
            pltpu.VMEM((N_CHUNKS, IDX_CHUNK), jnp.int32),
            pltpu.VMEM((N_CHUNKS, IDX_CHUNK), jnp.int32),
            pltpu.VMEM((B_PER_W, D), jnp.float32),
            pltpu.VMEM((B_PER_W, D), jnp.float32),
            pltpu.VMEM((B_PER_W, D), jnp.float32),
            pltpu.VMEM((B_PER_W,), jnp.float32),
            pltpu.SemaphoreType.DMA,
        ],
        compiler_params=pltpu.CompilerParams(needs_layout_passes=False,
                                             use_tc_tiling_on_sc=False),
    )(si, pi, oi, nodes, relations)


def kernel(si, pi, oi, nodes, relations):
    return _distmult(si.astype(jnp.int32), pi.astype(jnp.int32),
                     oi.astype(jnp.int32), nodes, relations)
